# baseline (speedup 1.0000x reference)
import functools

import jax
import jax.numpy as jnp
from jax import lax
from jax.experimental import pallas as pl
from jax.experimental.pallas import tpu as pltpu
from jax.experimental.pallas import tpu_sc as plsc

NQ = 256
DM = 1024
BS = 8


@functools.cache
def _build_sc_broadcast():
    info = plsc.get_sparse_core_info()
    nc, ns = info.num_cores, info.num_subcores
    nw = nc * ns
    rows = NQ // nw

    mesh = plsc.VectorSubcoreMesh(core_axis_name="c", subcore_axis_name="s")

    @functools.partial(
        pl.kernel,
        mesh=mesh,
        out_type=jax.ShapeDtypeStruct((BS * NQ, DM), jnp.float32),
        scratch_types=[
            pltpu.VMEM((rows, DM), jnp.float32),
            pltpu.SemaphoreType.DMA,
        ],
    )
    def broadcast_rows(qf_hbm, out_hbm, buf, sem):
        wid = lax.axis_index("s") * nc + lax.axis_index("c")
        qbase = wid * rows
        pltpu.sync_copy(qf_hbm.at[pl.ds(qbase, rows)], buf)
        copies = [
            pltpu.async_copy(buf, out_hbm.at[pl.ds(b * NQ + qbase, rows)], sem)
            for b in range(BS)
        ]
        for c in copies:
            c.wait()

    return broadcast_rows


def kernel(input_features, query_feat):
    bs = input_features.shape[1]
    out = _build_sc_broadcast()(query_feat)
    return out.reshape(bs, NQ, DM)
